# Initial kernel scaffold; baseline (speedup 1.0000x reference)
import jax
import jax.numpy as jnp
from jax.experimental import pallas as pl


def kernel(x, edge_index, edge_weight, params):
    raise NotImplementedError("write your pallas kernel here")


# speedup vs baseline: 1.0009x; 1.0009x over previous
import functools

import jax
import jax.numpy as jnp
from jax.experimental import pallas as pl


def _leaky(x):
    return jnp.where(x >= 0, x, 0.2 * x)


def _edge_feat5(h, src, dst, w, eps=1e-8):
    hs = h[src]
    hd = h[dst]
    ns = jnp.linalg.norm(hs, axis=1)
    nd = jnp.linalg.norm(hd, axis=1)
    dot = jnp.sum(hs * hd, axis=1)
    cos = dot / (ns * nd + eps)
    diff = jnp.linalg.norm(hs - hd, axis=1) / (h.shape[1] ** 0.5)
    return jnp.stack([cos, diff, ns, nd, w], axis=1)


def _edge_mlp(p, feat, mode):
    y = feat @ p['W1'] + p['b1']
    y = _leaky(y)
    y = y @ p['W2'] + p['b2']
    if mode == 'sigmoid':
        return jax.nn.sigmoid(y)
    return jax.nn.softplus(y)


def _gatv2(p, h, src, dst, edge_attr, n_nodes):
    xl = h @ p['Wl'] + p['bl']
    xr = h @ p['Wr'] + p['br']
    m = xl[src] + xr[dst] + edge_attr @ p['We']
    m = _leaky(m)
    a = jnp.sum(m * p['att'][None, :], axis=1)
    amax = jax.ops.segment_max(a, dst, num_segments=n_nodes)
    ae = jnp.exp(a - amax[dst])
    denom = jax.ops.segment_sum(ae, dst, num_segments=n_nodes)
    alpha = ae / (denom[dst] + 1e-16)
    out = jax.ops.segment_sum(xl[src] * alpha[:, None], dst, num_segments=n_nodes)
    return out + p['bias']


def _global_prune(w, ratio):
    E = w.shape[0]
    k = int(ratio * E)
    if ratio <= 0.0 or E < 8 or k <= 0:
        return w
    if k >= E:
        return jnp.zeros_like(w)
    kth = jnp.sort(w)[k - 1]
    return jnp.where(w >= kth, w, jnp.zeros_like(w))


def _ln_leaky_body(h_ref, g_ref, b_ref, o_ref):
    h = h_ref[...]
    mu = jnp.mean(h, axis=-1, keepdims=True)
    var = jnp.mean((h - mu) ** 2, axis=-1, keepdims=True)
    y = (h - mu) / jnp.sqrt(var + 1e-5) * g_ref[...] + b_ref[...]
    o_ref[...] = jnp.where(y >= 0, y, 0.2 * y)


def _ln_leaky(h, g, b):
    n, d = h.shape
    blk = 256
    grid = (n + blk - 1) // blk
    return pl.pallas_call(
        _ln_leaky_body,
        grid=(grid,),
        in_specs=[
            pl.BlockSpec((blk, d), lambda i: (i, 0)),
            pl.BlockSpec((1, d), lambda i: (0, 0)),
            pl.BlockSpec((1, d), lambda i: (0, 0)),
        ],
        out_specs=pl.BlockSpec((blk, d), lambda i: (i, 0)),
        out_shape=jax.ShapeDtypeStruct((n, d), h.dtype),
    )(h, g.reshape(1, d), b.reshape(1, d))


def kernel(x, edge_index, edge_weight, params):
    src = edge_index[0]
    dst = edge_index[1]
    n_nodes = x.shape[0]
    w_sp0 = edge_weight[:, 0]
    w_ft0 = edge_weight[:, 1]
    lambda_patch, beta_cross, prune_ratio_sp = 0.1, 0.2, 0.1
    h = x
    for lp in params['layers']:
        feat_sp = _edge_feat5(h, src, dst, w_sp0)
        feat_ft = _edge_feat5(h, src, dst, w_ft0)
        r_sp = _edge_mlp(lp['calib_sp'], feat_sp, 'sigmoid')[:, 0]
        r_ft = _edge_mlp(lp['calib_ft'], feat_ft, 'sigmoid')[:, 0]
        w_sp = w_sp0 * r_sp
        w_ft = w_ft0 * r_ft
        patch = _edge_mlp(lp['patch_ft'], feat_ft, 'softplus')[:, 0]
        w_ft = w_ft + lambda_patch * patch
        w_sp = _global_prune(w_sp, prune_ratio_sp)
        ea_sp = jnp.stack([w_sp, jnp.zeros_like(w_sp)], axis=1)
        ea_ft = jnp.stack([jnp.zeros_like(w_ft), w_ft], axis=1)
        u_sp = _gatv2(lp['conv_sp'], h, src, dst, ea_sp, n_nodes)
        u_ft = _gatv2(lp['conv_ft'], h, src, dst, ea_ft, n_nodes)
        u_sp = u_sp + beta_cross * ((u_ft - u_sp) @ lp['cross_sp']['W'] + lp['cross_sp']['b'])
        u_ft = u_ft + beta_cross * ((u_sp - u_ft) @ lp['cross_ft']['W'] + lp['cross_ft']['b'])
        h = (h @ lp['skip']['W'] + lp['skip']['b']) + u_sp + u_ft
        h = _ln_leaky(h, lp['norm']['g'], lp['norm']['b'])
    return h


# speedup vs baseline: 1.0055x; 1.0046x over previous
import functools

import jax
import jax.numpy as jnp
from jax import lax
from jax.experimental import pallas as pl
from jax.experimental.pallas import tpu as pltpu
from jax.experimental.pallas import tpu_sc as plsc

_NC, _NS, _L = 2, 16, 16
_NW = _NC * _NS


def _vsqrt(x):
    i = plsc.bitcast(x, jnp.int32)
    y = plsc.bitcast((i >> 1) + 0x1FBD1DF5, jnp.float32)
    for _ in range(3):
        y = 0.5 * (y + x / y)
    return y


def _sc_leaky(x):
    return jnp.maximum(x, 0.2 * x)


def _mlp16(f0, f1, f2, f3, f4, w1_ref, b1_ref, w2_ref, b2_ref, mi):
    feats = (f0, f1, f2, f3, f4)
    b2v = b2_ref[...]
    out = jnp.zeros((_L,), jnp.float32) + b2v[mi]
    for j16 in range(4):
        b1v = b1_ref[pl.ds(mi * 64 + j16 * _L, _L)]
        w2v = w2_ref[pl.ds(mi * 64 + j16 * _L, _L)]
        w1vs = [w1_ref[pl.ds((mi * 5 + i) * 64 + j16 * _L, _L)]
                for i in range(5)]
        for jj in range(_L):
            t = feats[0] * w1vs[0][jj] + b1v[jj]
            for i in range(1, 5):
                t = t + feats[i] * w1vs[i][jj]
            t = _sc_leaky(t)
            out = out + w2v[jj] * t
    return out


def _edge_mlp_sc(h, src_s, dst_s, w_sp0_s, w_ft0_s, w1c, b1c, w2c, b2c):
    n, d = h.shape
    e = src_s.shape[0]
    assert e % (_NW * 8) == 0
    epw = e // _NW
    nb = (epw + _L - 1) // _L
    epw_pad = nb * _L
    inv_sqrt_d = 1.0 / (d ** 0.5)

    mesh = plsc.VectorSubcoreMesh(core_axis_name="c", subcore_axis_name="s", num_cores=_NC, num_subcores=_NS)

    @functools.partial(
        pl.kernel,
        out_type=(
            jax.ShapeDtypeStruct((e,), jnp.float32),
            jax.ShapeDtypeStruct((e,), jnp.float32),
            jax.ShapeDtypeStruct((e,), jnp.float32),
        ),
        mesh=mesh,
        compiler_params=pltpu.CompilerParams(needs_layout_passes=False),
        scratch_types=[
            pltpu.VMEM((epw_pad,), jnp.int32),
            pltpu.VMEM((epw_pad,), jnp.int32),
            pltpu.VMEM((epw_pad,), jnp.float32),
            pltpu.VMEM((epw_pad,), jnp.float32),
            pltpu.VMEM((_L, d), jnp.float32),
            pltpu.VMEM((_L, d), jnp.float32),
            pltpu.VMEM((_L,), jnp.int32),
            pltpu.VMEM((_L,), jnp.int32),
            pltpu.VMEM((960,), jnp.float32),
            pltpu.VMEM((192,), jnp.float32),
            pltpu.VMEM((192,), jnp.float32),
            pltpu.VMEM((_L,), jnp.float32),
            pltpu.VMEM((epw_pad,), jnp.float32),
            pltpu.VMEM((epw_pad,), jnp.float32),
            pltpu.VMEM((epw_pad,), jnp.float32),
            pltpu.SemaphoreType.DMA,
            pltpu.SemaphoreType.DMA,
        ],
    )
    def k(h_hbm, src_hbm, dst_hbm, wsp_hbm, wft_hbm, w1_hbm, b1_hbm, w2_hbm,
          b2_hbm, osp_hbm, oft_hbm, opatch_hbm,
          src_v, dst_v, wsp_v, wft_v, hs_v, hd_v, bsi_v, bdi_v,
          w1_v, b1_v, w2_v, b2_v,
          osp_v, oft_v, opatch_v, sem1, sem2):
        wid = lax.axis_index("s") * _NC + lax.axis_index("c")
        base = pl.multiple_of(wid * epw, 8)
        zi = jnp.zeros((_L,), jnp.int32)
        if epw_pad > epw:
            src_v[pl.ds(epw_pad - _L, _L)] = zi
            dst_v[pl.ds(epw_pad - _L, _L)] = zi
        pltpu.sync_copy(src_hbm.at[pl.ds(base, epw)], src_v.at[pl.ds(0, epw)])
        pltpu.sync_copy(dst_hbm.at[pl.ds(base, epw)], dst_v.at[pl.ds(0, epw)])
        pltpu.sync_copy(wsp_hbm.at[pl.ds(base, epw)], wsp_v.at[pl.ds(0, epw)])
        pltpu.sync_copy(wft_hbm.at[pl.ds(base, epw)], wft_v.at[pl.ds(0, epw)])
        pltpu.sync_copy(w1_hbm, w1_v)
        pltpu.sync_copy(b1_hbm, b1_v)
        pltpu.sync_copy(w2_hbm, w2_v)
        pltpu.sync_copy(b2_hbm, b2_v)

        def batch(b, _):
            off = b * _L
            bsi_v[...] = src_v[pl.ds(off, _L)]
            bdi_v[...] = dst_v[pl.ds(off, _L)]
            pltpu.async_copy(h_hbm.at[bsi_v], hs_v, sem1)
            pltpu.async_copy(h_hbm.at[bdi_v], hd_v, sem2)
            pltpu.make_async_copy(h_hbm.at[bsi_v], hs_v, sem1).wait()
            pltpu.make_async_copy(h_hbm.at[bdi_v], hd_v, sem2).wait()
            lane = lax.iota(jnp.int32, _L)
            z = jnp.zeros((_L,), jnp.float32)
            dot, ss, dd = z, z, z
            for j in range(_L):
                def cbody(c, carry):
                    dacc, sacc, tacc = carry
                    a = hs_v[j, pl.ds(c * _L, _L)]
                    bb = hd_v[j, pl.ds(c * _L, _L)]
                    return (dacc + a * bb, sacc + a * a, tacc + bb * bb)

                dacc, sacc, tacc = lax.fori_loop(0, d // _L, cbody, (z, z, z))
                sel = lane == j
                dot = jnp.where(sel, jnp.sum(dacc), dot)
                ss = jnp.where(sel, jnp.sum(sacc), ss)
                dd = jnp.where(sel, jnp.sum(tacc), dd)
            ns = _vsqrt(ss)
            nd = _vsqrt(dd)
            cos = dot / (ns * nd + 1e-8)
            diff = _vsqrt(jnp.maximum(ss + dd - 2.0 * dot, 0.0)) * inv_sqrt_d
            wsp = wsp_v[pl.ds(off, _L)]
            wft = wft_v[pl.ds(off, _L)]
            y_sp = _mlp16(cos, diff, ns, nd, wsp, w1_v, b1_v, w2_v, b2_v, 0)
            y_ft = _mlp16(cos, diff, ns, nd, wft, w1_v, b1_v, w2_v, b2_v, 1)
            y_pt = _mlp16(cos, diff, ns, nd, wft, w1_v, b1_v, w2_v, b2_v, 2)
            osp_v[pl.ds(off, _L)] = wsp / (1.0 + jnp.exp(-y_sp))
            oft_v[pl.ds(off, _L)] = wft / (1.0 + jnp.exp(-y_ft))
            opatch_v[pl.ds(off, _L)] = y_pt
            return _

        lax.fori_loop(0, nb, batch, None)
        pltpu.sync_copy(osp_v.at[pl.ds(0, epw)], osp_hbm.at[pl.ds(base, epw)])
        pltpu.sync_copy(oft_v.at[pl.ds(0, epw)], oft_hbm.at[pl.ds(base, epw)])
        pltpu.sync_copy(opatch_v.at[pl.ds(0, epw)],
                        opatch_hbm.at[pl.ds(base, epw)])

    return k(h, src_s, dst_s, w_sp0_s, w_ft0_s, w1c, b1c, w2c, b2c)


def _gat_conv_sc(xl, xr, w_e, src_pad, rp_pad, wev, att):
    n, dout = xl.shape
    npw = 320
    assert _NW * npw >= n
    nck = dout // _L

    mesh = plsc.VectorSubcoreMesh(core_axis_name="c", subcore_axis_name="s", num_cores=_NC, num_subcores=_NS)

    @functools.partial(
        pl.kernel,
        out_type=jax.ShapeDtypeStruct((n, dout), jnp.float32),
        mesh=mesh,
        compiler_params=pltpu.CompilerParams(needs_layout_passes=False),
        scratch_types=[
            pltpu.VMEM((npw + _L,), jnp.int32),
            pltpu.VMEM((1, dout), jnp.float32),
            pltpu.VMEM((dout,), jnp.float32),
            pltpu.VMEM((dout,), jnp.float32),
            pltpu.VMEM((dout,), jnp.float32),
            pltpu.VMEM((1, dout), jnp.float32),
            pltpu.VMEM((_L,), jnp.int32),
            pltpu.VMEM((_L,), jnp.float32),
            pltpu.VMEM((_L, dout), jnp.float32),
            pltpu.SemaphoreType.DMA,
        ],
    )
    def k(xl_hbm, xr_hbm, we_hbm, src_hbm, rp_hbm, wev_hbm, att_hbm, out_hbm,
          rp_v, xr_v, att_v, wev_v, acc_v, out_v, sidx_v, wv_v, rows_v, sem):
        wid = lax.axis_index("s") * _NC + lax.axis_index("c")
        n_lo = pl.multiple_of(wid * npw, 8)
        pltpu.sync_copy(att_hbm, att_v)
        pltpu.sync_copy(wev_hbm, wev_v)
        pltpu.sync_copy(rp_hbm.at[pl.ds(n_lo, npw + _L)], rp_v)
        zf = jnp.zeros((_L,), jnp.float32)

        def node_body(ni, _):
            node = n_lo + ni

            @pl.when(node < n)
            def _process():
                rpv = rp_v[pl.ds(ni, _L)]
                e0 = rpv[0]
                e1 = rpv[1]
                pltpu.sync_copy(xr_hbm.at[pl.ds(node, 1)], xr_v)

                def zb(kk, carry):
                    acc_v[pl.ds(kk * _L, _L)] = zf
                    return carry

                lax.fori_loop(0, nck, zb, None)
                e0a = e0 & ~(_L - 1)
                nbat = (e1 - e0a + _L - 1) // _L

                def ebatch(b, carry):
                    m_v, s_v = carry
                    gb = pl.multiple_of(e0a + b * _L, _L)
                    pltpu.sync_copy(src_hbm.at[pl.ds(gb, _L)], sidx_v)
                    pltpu.sync_copy(we_hbm.at[pl.ds(gb, _L)], wv_v)
                    pltpu.async_copy(xl_hbm.at[sidx_v], rows_v, sem)
                    pltpu.make_async_copy(xl_hbm.at[sidx_v], rows_v, sem).wait()
                    wv = wv_v[...]
                    for j in range(_L):
                        ej = gb + j
                        valid = jnp.logical_and(ej >= e0, ej < e1)

                        def _true(carry_in, j=j):
                            m_in, s_in = carry_in
                            wj = wv[j]

                            def abody(kk, aacc):
                                sl = pl.ds(kk * _L, _L)
                                t = rows_v[j, sl] + xr_v[0, sl]
                                t = t + wj * wev_v[sl]
                                t = jnp.maximum(t, 0.2 * t)
                                return aacc + att_v[sl] * t

                            av = lax.fori_loop(0, nck, abody, zf)
                            a_v = jnp.zeros((_L,), jnp.float32) + jnp.sum(av)
                            mn_v = jnp.maximum(m_in, a_v)
                            c_v = jnp.exp(m_in - mn_v)
                            t_v = jnp.exp(a_v - mn_v)

                            def ubody(kk, carry2):
                                sl = pl.ds(kk * _L, _L)
                                acc_v[sl] = acc_v[sl] * c_v + t_v * rows_v[j, sl]
                                return carry2

                            lax.fori_loop(0, nck, ubody, None)
                            return (mn_v, s_in * c_v + t_v)

                        def _false(carry_in):
                            return carry_in

                        m_v, s_v = lax.cond(valid, _true, _false, (m_v, s_v))
                    return (m_v, s_v)

                m0 = jnp.full((_L,), -1e30, jnp.float32)
                _, s_v = lax.fori_loop(0, nbat, ebatch, (m0, zf))
                inv_v = 1.0 / (s_v + 1e-16)

                def ob(kk, carry):
                    sl = pl.ds(kk * _L, _L)
                    out_v[0, sl] = acc_v[sl] * inv_v
                    return carry

                lax.fori_loop(0, nck, ob, None)
                pltpu.sync_copy(out_v, out_hbm.at[pl.ds(node, 1)])

            return _

        lax.fori_loop(0, npw, node_body, None)

    return k(xl, xr, w_e, src_pad, rp_pad, wev, att)


def _kth_select_sc(w_pad, k_rank, e):
    epw = e // _NS
    assert epw % _L == 0

    mesh = plsc.VectorSubcoreMesh(core_axis_name="c", subcore_axis_name="s", num_cores=_NC, num_subcores=_NS)

    @functools.partial(
        pl.kernel,
        out_type=jax.ShapeDtypeStruct((_L,), jnp.float32),
        mesh=mesh,
        compiler_params=pltpu.CompilerParams(needs_layout_passes=False),
        scratch_types=[
            pltpu.VMEM((epw,), jnp.float32),
            pltpu.VMEM((1, _L), jnp.int32),
            pltpu.VMEM((_NS, _L), jnp.int32),
            pltpu.VMEM((_L,), jnp.float32),
            pltpu.VMEM_SHARED((_NS, _L), jnp.int32),
        ],
    )
    def k(w_hbm, out_hbm, wv_v, lcnt_v, allc_v, outst_v, shc_v):
        cid = lax.axis_index("c")
        sid = lax.axis_index("s")

        @pl.when(cid == 0)
        def _core0():
            pltpu.sync_copy(w_hbm.at[pl.ds(pl.multiple_of(sid * epw, 8), epw)], wv_v)
            zi = jnp.zeros((_L,), jnp.int32)
            p_v = zi

            for bit in range(30, -1, -1):
                p1_v = p_v | (1 << bit)

                def cb(i, acc):
                    uv = plsc.bitcast(wv_v[pl.ds(i * _L, _L)], jnp.int32)
                    return acc + jnp.where(uv < p1_v, 1, 0).astype(jnp.int32)

                lanes = lax.fori_loop(0, epw // _L, cb, zi)
                lcnt_v[0, pl.ds(0, _L)] = lanes
                pltpu.sync_copy(lcnt_v, shc_v.at[pl.ds(sid, 1)])
                plsc.subcore_barrier()
                pltpu.sync_copy(shc_v, allc_v)
                tot = jnp.zeros((_L,), jnp.int32)
                for r in range(_NS):
                    tot = tot + allc_v[r, pl.ds(0, _L)]
                total_v = jnp.zeros((_L,), jnp.int32) + jnp.sum(tot)
                p_v = jnp.where(total_v >= k_rank, p_v, p1_v)
                plsc.subcore_barrier()

            @pl.when(sid == 0)
            def _write():
                outst_v[...] = plsc.bitcast(p_v, jnp.float32)
                pltpu.sync_copy(outst_v, out_hbm)

    return k(w_pad)


def _tc_matmul(a, w, bias):
    n, kk = a.shape
    m = w.shape[1]
    bn = 256
    bm = 512 if m % 512 == 0 else 128
    grid = ((n + bn - 1) // bn, m // bm)

    def body(a_ref, w_ref, b_ref, o_ref):
        o_ref[...] = jnp.dot(a_ref[...], w_ref[...],
                             preferred_element_type=jnp.float32) + b_ref[...]

    return pl.pallas_call(
        body,
        grid=grid,
        in_specs=[
            pl.BlockSpec((bn, kk), lambda i, j: (i, 0)),
            pl.BlockSpec((kk, bm), lambda i, j: (0, j)),
            pl.BlockSpec((1, bm), lambda i, j: (0, j)),
        ],
        out_specs=pl.BlockSpec((bn, bm), lambda i, j: (i, j)),
        out_shape=jax.ShapeDtypeStruct((n, m), jnp.float32),
    )(a, w, bias.reshape(1, m))


def _tc_edge_fixup(w_sp_raw, kth16, w_ft_pre, y_patch, e):
    er, ec = 1250, 128
    assert er * ec == e

    def body(a_ref, k_ref, b_ref, c_ref, osp_ref, oft_ref):
        kth = k_ref[0, 0]
        a = a_ref[...]
        osp_ref[...] = jnp.where(a >= kth, a, 0.0)
        oft_ref[...] = b_ref[...] + 0.1 * jax.nn.softplus(c_ref[...])

    osp, oft = pl.pallas_call(
        body,
        grid=(1,),
        in_specs=[
            pl.BlockSpec((er, ec), lambda i: (0, 0)),
            pl.BlockSpec((1, 1), lambda i: (0, 0), memory_space=pltpu.SMEM),
            pl.BlockSpec((er, ec), lambda i: (0, 0)),
            pl.BlockSpec((er, ec), lambda i: (0, 0)),
        ],
        out_specs=[pl.BlockSpec((er, ec), lambda i: (0, 0)),
                   pl.BlockSpec((er, ec), lambda i: (0, 0))],
        out_shape=[jax.ShapeDtypeStruct((er, ec), jnp.float32),
                   jax.ShapeDtypeStruct((er, ec), jnp.float32)],
    )(w_sp_raw.reshape(er, ec), kth16[:1].reshape(1, 1),
      w_ft_pre.reshape(er, ec), y_patch.reshape(er, ec))
    return osp.reshape(e), oft.reshape(e)


def _tc_sub_bias(a, b, bias):
    n, m = a.shape
    bn = 256

    def body(a_ref, b_ref, bias_ref, o_ref):
        o_ref[...] = a_ref[...] - b_ref[...] + bias_ref[...]

    return pl.pallas_call(
        body,
        grid=((n + bn - 1) // bn,),
        in_specs=[
            pl.BlockSpec((bn, m), lambda i: (i, 0)),
            pl.BlockSpec((bn, m), lambda i: (i, 0)),
            pl.BlockSpec((1, m), lambda i: (0, 0)),
        ],
        out_specs=pl.BlockSpec((bn, m), lambda i: (i, 0)),
        out_shape=jax.ShapeDtypeStruct((n, m), jnp.float32),
    )(a, b, bias.reshape(1, m))


def _tc_scale_sub(a, b):
    n, m = a.shape
    bn = 256

    def body(a_ref, b_ref, o_ref):
        o_ref[...] = 0.2 * a_ref[...] - b_ref[...]

    return pl.pallas_call(
        body,
        grid=((n + bn - 1) // bn,),
        in_specs=[
            pl.BlockSpec((bn, m), lambda i: (i, 0)),
            pl.BlockSpec((bn, m), lambda i: (i, 0)),
        ],
        out_specs=pl.BlockSpec((bn, m), lambda i: (i, 0)),
        out_shape=jax.ShapeDtypeStruct((n, m), jnp.float32),
    )(a, b)


def _tc_final(skip, osp, oft, cs1, cs2, bias_sum, g, bvec):
    n, m = skip.shape
    bn = 256

    def body(s_ref, a_ref, b_ref, c_ref, d_ref, bb_ref, g_ref, b2_ref, o_ref):
        tot = (s_ref[...] + a_ref[...] + b_ref[...]
               + 0.2 * (c_ref[...] + d_ref[...]) + bb_ref[...])
        mu = jnp.mean(tot, axis=-1, keepdims=True)
        var = jnp.mean((tot - mu) ** 2, axis=-1, keepdims=True)
        y = (tot - mu) / jnp.sqrt(var + 1e-5) * g_ref[...] + b2_ref[...]
        o_ref[...] = jnp.where(y >= 0, y, 0.2 * y)

    return pl.pallas_call(
        body,
        grid=((n + bn - 1) // bn,),
        in_specs=[pl.BlockSpec((bn, m), lambda i: (i, 0))] * 5
        + [pl.BlockSpec((1, m), lambda i: (0, 0))] * 3,
        out_specs=pl.BlockSpec((bn, m), lambda i: (i, 0)),
        out_shape=jax.ShapeDtypeStruct((n, m), jnp.float32),
    )(skip, osp, oft, cs1, cs2, bias_sum.reshape(1, m), g.reshape(1, m),
      bvec.reshape(1, m))


def kernel(x, edge_index, edge_weight, params):
    n_nodes = x.shape[0]
    e = edge_index.shape[1]
    npw = 320
    perm = jnp.argsort(edge_index[1])
    src = edge_index[0][perm].astype(jnp.int32)
    dst = edge_index[1][perm].astype(jnp.int32)
    w_sp0 = edge_weight[perm, 0]
    w_ft0 = edge_weight[perm, 1]
    rp_pad = jnp.searchsorted(
        dst, jnp.arange(_NW * npw + _L + 1, dtype=jnp.int32), side='left'
    ).astype(jnp.int32)
    src_pad = jnp.concatenate([src, jnp.zeros((_L,), jnp.int32)])
    padf = jnp.zeros((_L,), jnp.float32)
    k_rank = int(0.1 * e)
    lambda_patch = 0.1
    h = x
    for lp in params['layers']:
        dout = lp['skip']['W'].shape[1]
        w1c = jnp.stack([lp['calib_sp']['W1'], lp['calib_ft']['W1'],
                         lp['patch_ft']['W1']])
        b1c = jnp.stack([lp['calib_sp']['b1'], lp['calib_ft']['b1'],
                         lp['patch_ft']['b1']])
        w2c = jnp.stack([lp['calib_sp']['W2'][:, 0], lp['calib_ft']['W2'][:, 0],
                         lp['patch_ft']['W2'][:, 0]])
        b2c = jnp.concatenate([lp['calib_sp']['b2'], lp['calib_ft']['b2'],
                               lp['patch_ft']['b2'],
                               jnp.zeros((13,), jnp.float32)])
        w_sp_raw, w_ft_pre, y_patch = _edge_mlp_sc(
            h, src, dst, w_sp0, w_ft0, w1c.reshape(-1), b1c.reshape(-1),
            w2c.reshape(-1), b2c)
        kth16 = _kth_select_sc(w_sp_raw, k_rank, e)
        w_sp, w_ft = _tc_edge_fixup(w_sp_raw, kth16, w_ft_pre, y_patch, e)

        psp, pft, ps = lp['conv_sp'], lp['conv_ft'], lp['skip']
        wcat = jnp.concatenate(
            [psp['Wl'], psp['Wr'], pft['Wl'], pft['Wr'], ps['W']], axis=1)
        bcat = jnp.concatenate(
            [psp['bl'], psp['br'], pft['bl'], pft['br'], ps['b']])
        big = _tc_matmul(h, wcat, bcat)
        xl_sp = big[:, 0:dout]
        xr_sp = big[:, dout:2 * dout]
        xl_ft = big[:, 2 * dout:3 * dout]
        xr_ft = big[:, 3 * dout:4 * dout]
        skip_pre = big[:, 4 * dout:5 * dout]

        o_sp = _gat_conv_sc(xl_sp, xr_sp, jnp.concatenate([w_sp, padf]),
                            src_pad, rp_pad, psp['We'][0], psp['att'])
        o_ft = _gat_conv_sc(xl_ft, xr_ft, jnp.concatenate([w_ft, padf]),
                            src_pad, rp_pad, pft['We'][1], pft['att'])

        d1 = _tc_sub_bias(o_ft, o_sp, pft['bias'] - psp['bias'])
        cs1 = _tc_matmul(d1, lp['cross_sp']['W'], lp['cross_sp']['b'])
        d2 = _tc_scale_sub(cs1, d1)
        cs2 = _tc_matmul(d2, lp['cross_ft']['W'], lp['cross_ft']['b'])
        h = _tc_final(skip_pre, o_sp, o_ft, cs1, cs2,
                      psp['bias'] + pft['bias'],
                      lp['norm']['g'], lp['norm']['b'])
    return h


# speedup vs baseline: 1.1162x; 1.1101x over previous
import functools

import jax
import jax.numpy as jnp
from jax import lax
from jax.experimental import pallas as pl
from jax.experimental.pallas import tpu as pltpu
from jax.experimental.pallas import tpu_sc as plsc

_NC, _NS, _L = 2, 16, 16
_NW = _NC * _NS


def _vsqrt(x):
    i = plsc.bitcast(x, jnp.int32)
    y = plsc.bitcast((i >> 1) + 0x1FBD1DF5, jnp.float32)
    for _ in range(3):
        y = 0.5 * (y + x / y)
    return y


def _sc_leaky(x):
    return jnp.maximum(x, 0.2 * x)


def _mlp16(f0, f1, f2, f3, f4, w1_ref, b1_ref, w2_ref, b2_ref, mi):
    feats = (f0, f1, f2, f3, f4)
    b2v = b2_ref[...]
    out = jnp.zeros((_L,), jnp.float32) + b2v[mi]
    for j16 in range(4):
        b1v = b1_ref[pl.ds(mi * 64 + j16 * _L, _L)]
        w2v = w2_ref[pl.ds(mi * 64 + j16 * _L, _L)]
        w1vs = [w1_ref[pl.ds((mi * 5 + i) * 64 + j16 * _L, _L)]
                for i in range(5)]
        for jj in range(_L):
            t = feats[0] * w1vs[0][jj] + b1v[jj]
            for i in range(1, 5):
                t = t + feats[i] * w1vs[i][jj]
            t = _sc_leaky(t)
            out = out + w2v[jj] * t
    return out


def _edge_mlp_sc(h, src_s, dst_s, w_sp0_s, w_ft0_s, w1c, b1c, w2c, b2c):
    n, d = h.shape
    e = src_s.shape[0]
    assert e % (_NW * 8) == 0
    epw = e // _NW
    nb = (epw + _L - 1) // _L
    epw_pad = nb * _L
    inv_sqrt_d = 1.0 / (d ** 0.5)

    mesh = plsc.VectorSubcoreMesh(core_axis_name="c", subcore_axis_name="s", num_cores=_NC, num_subcores=_NS)

    @functools.partial(
        pl.kernel,
        out_type=(
            jax.ShapeDtypeStruct((e,), jnp.float32),
            jax.ShapeDtypeStruct((e,), jnp.float32),
            jax.ShapeDtypeStruct((e,), jnp.float32),
        ),
        mesh=mesh,
        compiler_params=pltpu.CompilerParams(needs_layout_passes=False),
        scratch_types=[
            pltpu.VMEM((epw_pad,), jnp.int32),
            pltpu.VMEM((epw_pad,), jnp.int32),
            pltpu.VMEM((epw_pad,), jnp.float32),
            pltpu.VMEM((epw_pad,), jnp.float32),
            pltpu.VMEM((_L, d), jnp.float32),
            pltpu.VMEM((_L, d), jnp.float32),
            pltpu.VMEM((_L,), jnp.int32),
            pltpu.VMEM((_L,), jnp.int32),
            pltpu.VMEM((960,), jnp.float32),
            pltpu.VMEM((192,), jnp.float32),
            pltpu.VMEM((192,), jnp.float32),
            pltpu.VMEM((_L,), jnp.float32),
            pltpu.VMEM((epw_pad,), jnp.float32),
            pltpu.VMEM((epw_pad,), jnp.float32),
            pltpu.VMEM((epw_pad,), jnp.float32),
            pltpu.SemaphoreType.DMA,
            pltpu.SemaphoreType.DMA,
        ],
    )
    def k(h_hbm, src_hbm, dst_hbm, wsp_hbm, wft_hbm, w1_hbm, b1_hbm, w2_hbm,
          b2_hbm, osp_hbm, oft_hbm, opatch_hbm,
          src_v, dst_v, wsp_v, wft_v, hs_v, hd_v, bsi_v, bdi_v,
          w1_v, b1_v, w2_v, b2_v,
          osp_v, oft_v, opatch_v, sem1, sem2):
        wid = lax.axis_index("s") * _NC + lax.axis_index("c")
        base = pl.multiple_of(wid * epw, 8)
        zi = jnp.zeros((_L,), jnp.int32)
        if epw_pad > epw:
            src_v[pl.ds(epw_pad - _L, _L)] = zi
            dst_v[pl.ds(epw_pad - _L, _L)] = zi
        pltpu.sync_copy(src_hbm.at[pl.ds(base, epw)], src_v.at[pl.ds(0, epw)])
        pltpu.sync_copy(dst_hbm.at[pl.ds(base, epw)], dst_v.at[pl.ds(0, epw)])
        pltpu.sync_copy(wsp_hbm.at[pl.ds(base, epw)], wsp_v.at[pl.ds(0, epw)])
        pltpu.sync_copy(wft_hbm.at[pl.ds(base, epw)], wft_v.at[pl.ds(0, epw)])
        pltpu.sync_copy(w1_hbm, w1_v)
        pltpu.sync_copy(b1_hbm, b1_v)
        pltpu.sync_copy(w2_hbm, w2_v)
        pltpu.sync_copy(b2_hbm, b2_v)

        def batch(b, _):
            off = b * _L
            bsi_v[...] = src_v[pl.ds(off, _L)]
            bdi_v[...] = dst_v[pl.ds(off, _L)]
            pltpu.async_copy(h_hbm.at[bsi_v], hs_v, sem1)
            pltpu.async_copy(h_hbm.at[bdi_v], hd_v, sem2)
            pltpu.make_async_copy(h_hbm.at[bsi_v], hs_v, sem1).wait()
            pltpu.make_async_copy(h_hbm.at[bdi_v], hd_v, sem2).wait()
            lane = lax.iota(jnp.int32, _L)
            z = jnp.zeros((_L,), jnp.float32)
            dot, ss, dd = z, z, z
            for j in range(_L):
                def cbody(c, carry):
                    dacc, sacc, tacc = carry
                    a = hs_v[j, pl.ds(c * _L, _L)]
                    bb = hd_v[j, pl.ds(c * _L, _L)]
                    return (dacc + a * bb, sacc + a * a, tacc + bb * bb)

                dacc, sacc, tacc = lax.fori_loop(0, d // _L, cbody, (z, z, z), unroll=8)
                sel = lane == j
                dot = jnp.where(sel, jnp.sum(dacc), dot)
                ss = jnp.where(sel, jnp.sum(sacc), ss)
                dd = jnp.where(sel, jnp.sum(tacc), dd)
            ns = _vsqrt(ss)
            nd = _vsqrt(dd)
            cos = dot / (ns * nd + 1e-8)
            diff = _vsqrt(jnp.maximum(ss + dd - 2.0 * dot, 0.0)) * inv_sqrt_d
            wsp = wsp_v[pl.ds(off, _L)]
            wft = wft_v[pl.ds(off, _L)]
            y_sp = _mlp16(cos, diff, ns, nd, wsp, w1_v, b1_v, w2_v, b2_v, 0)
            y_ft = _mlp16(cos, diff, ns, nd, wft, w1_v, b1_v, w2_v, b2_v, 1)
            y_pt = _mlp16(cos, diff, ns, nd, wft, w1_v, b1_v, w2_v, b2_v, 2)
            osp_v[pl.ds(off, _L)] = wsp / (1.0 + jnp.exp(-y_sp))
            oft_v[pl.ds(off, _L)] = wft / (1.0 + jnp.exp(-y_ft))
            opatch_v[pl.ds(off, _L)] = y_pt
            return _

        lax.fori_loop(0, nb, batch, None)
        pltpu.sync_copy(osp_v.at[pl.ds(0, epw)], osp_hbm.at[pl.ds(base, epw)])
        pltpu.sync_copy(oft_v.at[pl.ds(0, epw)], oft_hbm.at[pl.ds(base, epw)])
        pltpu.sync_copy(opatch_v.at[pl.ds(0, epw)],
                        opatch_hbm.at[pl.ds(base, epw)])

    return k(h, src_s, dst_s, w_sp0_s, w_ft0_s, w1c, b1c, w2c, b2c)


def _gat_conv_sc(xl, xr, w_e, src_pad, rp_pad, wev, att):
    n, dout = xl.shape
    npw = 320
    assert _NW * npw >= n
    nck = dout // _L

    mesh = plsc.VectorSubcoreMesh(core_axis_name="c", subcore_axis_name="s", num_cores=_NC, num_subcores=_NS)

    @functools.partial(
        pl.kernel,
        out_type=jax.ShapeDtypeStruct((n, dout), jnp.float32),
        mesh=mesh,
        compiler_params=pltpu.CompilerParams(needs_layout_passes=False),
        scratch_types=[
            pltpu.VMEM((npw + _L,), jnp.int32),
            pltpu.VMEM((1, dout), jnp.float32),
            pltpu.VMEM((dout,), jnp.float32),
            pltpu.VMEM((dout,), jnp.float32),
            pltpu.VMEM((dout,), jnp.float32),
            pltpu.VMEM((1, dout), jnp.float32),
            pltpu.VMEM((_L,), jnp.int32),
            pltpu.VMEM((_L,), jnp.float32),
            pltpu.VMEM((_L, dout), jnp.float32),
            pltpu.SemaphoreType.DMA,
        ],
    )
    def k(xl_hbm, xr_hbm, we_hbm, src_hbm, rp_hbm, wev_hbm, att_hbm, out_hbm,
          rp_v, xr_v, att_v, wev_v, acc_v, out_v, sidx_v, wv_v, rows_v, sem):
        wid = lax.axis_index("s") * _NC + lax.axis_index("c")
        n_lo = pl.multiple_of(wid * npw, 8)
        pltpu.sync_copy(att_hbm, att_v)
        pltpu.sync_copy(wev_hbm, wev_v)
        pltpu.sync_copy(rp_hbm.at[pl.ds(n_lo, npw + _L)], rp_v)
        zf = jnp.zeros((_L,), jnp.float32)

        def node_body(ni, _):
            node = n_lo + ni

            @pl.when(node < n)
            def _process():
                rpv = rp_v[pl.ds(ni, _L)]
                e0 = rpv[0]
                e1 = rpv[1]
                pltpu.sync_copy(xr_hbm.at[pl.ds(node, 1)], xr_v)

                def zb(kk, carry):
                    acc_v[pl.ds(kk * _L, _L)] = zf
                    return carry

                lax.fori_loop(0, nck, zb, None, unroll=8)
                e0a = e0 & ~(_L - 1)
                nbat = (e1 - e0a + _L - 1) // _L

                def ebatch(b, carry):
                    m_v, s_v = carry
                    gb = pl.multiple_of(e0a + b * _L, _L)
                    pltpu.sync_copy(src_hbm.at[pl.ds(gb, _L)], sidx_v)
                    pltpu.sync_copy(we_hbm.at[pl.ds(gb, _L)], wv_v)
                    pltpu.async_copy(xl_hbm.at[sidx_v], rows_v, sem)
                    pltpu.make_async_copy(xl_hbm.at[sidx_v], rows_v, sem).wait()
                    wv = wv_v[...]
                    for j in range(_L):
                        ej = gb + j
                        valid = jnp.logical_and(ej >= e0, ej < e1)

                        def _true(carry_in, j=j):
                            m_in, s_in = carry_in
                            wj = wv[j]

                            def abody(kk, aacc):
                                sl = pl.ds(kk * _L, _L)
                                t = rows_v[j, sl] + xr_v[0, sl]
                                t = t + wj * wev_v[sl]
                                t = jnp.maximum(t, 0.2 * t)
                                return aacc + att_v[sl] * t

                            av = lax.fori_loop(0, nck, abody, zf, unroll=8)
                            a_v = jnp.zeros((_L,), jnp.float32) + jnp.sum(av)
                            mn_v = jnp.maximum(m_in, a_v)
                            c_v = jnp.exp(m_in - mn_v)
                            t_v = jnp.exp(a_v - mn_v)

                            def ubody(kk, carry2):
                                sl = pl.ds(kk * _L, _L)
                                acc_v[sl] = acc_v[sl] * c_v + t_v * rows_v[j, sl]
                                return carry2

                            lax.fori_loop(0, nck, ubody, None, unroll=8)
                            return (mn_v, s_in * c_v + t_v)

                        def _false(carry_in):
                            return carry_in

                        m_v, s_v = lax.cond(valid, _true, _false, (m_v, s_v))
                    return (m_v, s_v)

                m0 = jnp.full((_L,), -1e30, jnp.float32)
                _, s_v = lax.fori_loop(0, nbat, ebatch, (m0, zf))
                inv_v = 1.0 / (s_v + 1e-16)

                def ob(kk, carry):
                    sl = pl.ds(kk * _L, _L)
                    out_v[0, sl] = acc_v[sl] * inv_v
                    return carry

                lax.fori_loop(0, nck, ob, None, unroll=8)
                pltpu.sync_copy(out_v, out_hbm.at[pl.ds(node, 1)])

            return _

        lax.fori_loop(0, npw, node_body, None)

    return k(xl, xr, w_e, src_pad, rp_pad, wev, att)


def _kth_select_sc(w_pad, k_rank, e):
    epw = e // _NS
    assert epw % _L == 0

    mesh = plsc.VectorSubcoreMesh(core_axis_name="c", subcore_axis_name="s", num_cores=_NC, num_subcores=_NS)

    @functools.partial(
        pl.kernel,
        out_type=jax.ShapeDtypeStruct((_L,), jnp.float32),
        mesh=mesh,
        compiler_params=pltpu.CompilerParams(needs_layout_passes=False),
        scratch_types=[
            pltpu.VMEM((epw,), jnp.float32),
            pltpu.VMEM((1, _L), jnp.int32),
            pltpu.VMEM((_NS, _L), jnp.int32),
            pltpu.VMEM((_L,), jnp.float32),
            pltpu.VMEM_SHARED((_NS, _L), jnp.int32),
        ],
    )
    def k(w_hbm, out_hbm, wv_v, lcnt_v, allc_v, outst_v, shc_v):
        cid = lax.axis_index("c")
        sid = lax.axis_index("s")

        @pl.when(cid == 0)
        def _core0():
            pltpu.sync_copy(w_hbm.at[pl.ds(pl.multiple_of(sid * epw, 8), epw)], wv_v)
            zi = jnp.zeros((_L,), jnp.int32)
            p_v = zi

            for bit in range(30, -1, -1):
                p1_v = p_v | (1 << bit)

                def cb(i, acc):
                    uv = plsc.bitcast(wv_v[pl.ds(i * _L, _L)], jnp.int32)
                    return acc + jnp.where(uv < p1_v, 1, 0).astype(jnp.int32)

                lanes = lax.fori_loop(0, epw // _L, cb, zi, unroll=8)
                lcnt_v[0, pl.ds(0, _L)] = lanes
                pltpu.sync_copy(lcnt_v, shc_v.at[pl.ds(sid, 1)])
                plsc.subcore_barrier()
                pltpu.sync_copy(shc_v, allc_v)
                tot = jnp.zeros((_L,), jnp.int32)
                for r in range(_NS):
                    tot = tot + allc_v[r, pl.ds(0, _L)]
                total_v = jnp.zeros((_L,), jnp.int32) + jnp.sum(tot)
                p_v = jnp.where(total_v >= k_rank, p_v, p1_v)
                plsc.subcore_barrier()

            @pl.when(sid == 0)
            def _write():
                outst_v[...] = plsc.bitcast(p_v, jnp.float32)
                pltpu.sync_copy(outst_v, out_hbm)

    return k(w_pad)


def _tc_matmul(a, w, bias):
    n, kk = a.shape
    m = w.shape[1]
    bn = 256
    bm = 512 if m % 512 == 0 else 128
    grid = ((n + bn - 1) // bn, m // bm)

    def body(a_ref, w_ref, b_ref, o_ref):
        o_ref[...] = jnp.dot(a_ref[...], w_ref[...],
                             preferred_element_type=jnp.float32) + b_ref[...]

    return pl.pallas_call(
        body,
        grid=grid,
        in_specs=[
            pl.BlockSpec((bn, kk), lambda i, j: (i, 0)),
            pl.BlockSpec((kk, bm), lambda i, j: (0, j)),
            pl.BlockSpec((1, bm), lambda i, j: (0, j)),
        ],
        out_specs=pl.BlockSpec((bn, bm), lambda i, j: (i, j)),
        out_shape=jax.ShapeDtypeStruct((n, m), jnp.float32),
    )(a, w, bias.reshape(1, m))


def _tc_edge_fixup(w_sp_raw, kth16, w_ft_pre, y_patch, e):
    er, ec = 1250, 128
    assert er * ec == e

    def body(a_ref, k_ref, b_ref, c_ref, osp_ref, oft_ref):
        kth = k_ref[0, 0]
        a = a_ref[...]
        osp_ref[...] = jnp.where(a >= kth, a, 0.0)
        oft_ref[...] = b_ref[...] + 0.1 * jax.nn.softplus(c_ref[...])

    osp, oft = pl.pallas_call(
        body,
        grid=(1,),
        in_specs=[
            pl.BlockSpec((er, ec), lambda i: (0, 0)),
            pl.BlockSpec((1, 1), lambda i: (0, 0), memory_space=pltpu.SMEM),
            pl.BlockSpec((er, ec), lambda i: (0, 0)),
            pl.BlockSpec((er, ec), lambda i: (0, 0)),
        ],
        out_specs=[pl.BlockSpec((er, ec), lambda i: (0, 0)),
                   pl.BlockSpec((er, ec), lambda i: (0, 0))],
        out_shape=[jax.ShapeDtypeStruct((er, ec), jnp.float32),
                   jax.ShapeDtypeStruct((er, ec), jnp.float32)],
    )(w_sp_raw.reshape(er, ec), kth16[:1].reshape(1, 1),
      w_ft_pre.reshape(er, ec), y_patch.reshape(er, ec))
    return osp.reshape(e), oft.reshape(e)


def _tc_sub_bias(a, b, bias):
    n, m = a.shape
    bn = 256

    def body(a_ref, b_ref, bias_ref, o_ref):
        o_ref[...] = a_ref[...] - b_ref[...] + bias_ref[...]

    return pl.pallas_call(
        body,
        grid=((n + bn - 1) // bn,),
        in_specs=[
            pl.BlockSpec((bn, m), lambda i: (i, 0)),
            pl.BlockSpec((bn, m), lambda i: (i, 0)),
            pl.BlockSpec((1, m), lambda i: (0, 0)),
        ],
        out_specs=pl.BlockSpec((bn, m), lambda i: (i, 0)),
        out_shape=jax.ShapeDtypeStruct((n, m), jnp.float32),
    )(a, b, bias.reshape(1, m))


def _tc_scale_sub(a, b):
    n, m = a.shape
    bn = 256

    def body(a_ref, b_ref, o_ref):
        o_ref[...] = 0.2 * a_ref[...] - b_ref[...]

    return pl.pallas_call(
        body,
        grid=((n + bn - 1) // bn,),
        in_specs=[
            pl.BlockSpec((bn, m), lambda i: (i, 0)),
            pl.BlockSpec((bn, m), lambda i: (i, 0)),
        ],
        out_specs=pl.BlockSpec((bn, m), lambda i: (i, 0)),
        out_shape=jax.ShapeDtypeStruct((n, m), jnp.float32),
    )(a, b)


def _tc_final(skip, osp, oft, cs1, cs2, bias_sum, g, bvec):
    n, m = skip.shape
    bn = 256

    def body(s_ref, a_ref, b_ref, c_ref, d_ref, bb_ref, g_ref, b2_ref, o_ref):
        tot = (s_ref[...] + a_ref[...] + b_ref[...]
               + 0.2 * (c_ref[...] + d_ref[...]) + bb_ref[...])
        mu = jnp.mean(tot, axis=-1, keepdims=True)
        var = jnp.mean((tot - mu) ** 2, axis=-1, keepdims=True)
        y = (tot - mu) / jnp.sqrt(var + 1e-5) * g_ref[...] + b2_ref[...]
        o_ref[...] = jnp.where(y >= 0, y, 0.2 * y)

    return pl.pallas_call(
        body,
        grid=((n + bn - 1) // bn,),
        in_specs=[pl.BlockSpec((bn, m), lambda i: (i, 0))] * 5
        + [pl.BlockSpec((1, m), lambda i: (0, 0))] * 3,
        out_specs=pl.BlockSpec((bn, m), lambda i: (i, 0)),
        out_shape=jax.ShapeDtypeStruct((n, m), jnp.float32),
    )(skip, osp, oft, cs1, cs2, bias_sum.reshape(1, m), g.reshape(1, m),
      bvec.reshape(1, m))


def kernel(x, edge_index, edge_weight, params):
    n_nodes = x.shape[0]
    e = edge_index.shape[1]
    npw = 320
    perm = jnp.argsort(edge_index[1])
    src = edge_index[0][perm].astype(jnp.int32)
    dst = edge_index[1][perm].astype(jnp.int32)
    w_sp0 = edge_weight[perm, 0]
    w_ft0 = edge_weight[perm, 1]
    rp_pad = jnp.searchsorted(
        dst, jnp.arange(_NW * npw + _L + 1, dtype=jnp.int32), side='left'
    ).astype(jnp.int32)
    src_pad = jnp.concatenate([src, jnp.zeros((_L,), jnp.int32)])
    padf = jnp.zeros((_L,), jnp.float32)
    k_rank = int(0.1 * e)
    lambda_patch = 0.1
    h = x
    for lp in params['layers']:
        dout = lp['skip']['W'].shape[1]
        w1c = jnp.stack([lp['calib_sp']['W1'], lp['calib_ft']['W1'],
                         lp['patch_ft']['W1']])
        b1c = jnp.stack([lp['calib_sp']['b1'], lp['calib_ft']['b1'],
                         lp['patch_ft']['b1']])
        w2c = jnp.stack([lp['calib_sp']['W2'][:, 0], lp['calib_ft']['W2'][:, 0],
                         lp['patch_ft']['W2'][:, 0]])
        b2c = jnp.concatenate([lp['calib_sp']['b2'], lp['calib_ft']['b2'],
                               lp['patch_ft']['b2'],
                               jnp.zeros((13,), jnp.float32)])
        w_sp_raw, w_ft_pre, y_patch = _edge_mlp_sc(
            h, src, dst, w_sp0, w_ft0, w1c.reshape(-1), b1c.reshape(-1),
            w2c.reshape(-1), b2c)
        kth16 = _kth_select_sc(w_sp_raw, k_rank, e)
        w_sp, w_ft = _tc_edge_fixup(w_sp_raw, kth16, w_ft_pre, y_patch, e)

        psp, pft, ps = lp['conv_sp'], lp['conv_ft'], lp['skip']
        wcat = jnp.concatenate(
            [psp['Wl'], psp['Wr'], pft['Wl'], pft['Wr'], ps['W']], axis=1)
        bcat = jnp.concatenate(
            [psp['bl'], psp['br'], pft['bl'], pft['br'], ps['b']])
        big = _tc_matmul(h, wcat, bcat)
        xl_sp = big[:, 0:dout]
        xr_sp = big[:, dout:2 * dout]
        xl_ft = big[:, 2 * dout:3 * dout]
        xr_ft = big[:, 3 * dout:4 * dout]
        skip_pre = big[:, 4 * dout:5 * dout]

        o_sp = _gat_conv_sc(xl_sp, xr_sp, jnp.concatenate([w_sp, padf]),
                            src_pad, rp_pad, psp['We'][0], psp['att'])
        o_ft = _gat_conv_sc(xl_ft, xr_ft, jnp.concatenate([w_ft, padf]),
                            src_pad, rp_pad, pft['We'][1], pft['att'])

        d1 = _tc_sub_bias(o_ft, o_sp, pft['bias'] - psp['bias'])
        cs1 = _tc_matmul(d1, lp['cross_sp']['W'], lp['cross_sp']['b'])
        d2 = _tc_scale_sub(cs1, d1)
        cs2 = _tc_matmul(d2, lp['cross_ft']['W'], lp['cross_ft']['b'])
        h = _tc_final(skip_pre, o_sp, o_ft, cs1, cs2,
                      psp['bias'] + pft['bias'],
                      lp['norm']['g'], lp['norm']['b'])
    return h
